# baseline (speedup 1.0000x reference)
import functools

import jax
import jax.numpy as jnp
from jax import lax
from jax.experimental import pallas as pl
from jax.experimental.pallas import tpu as pltpu
from jax.experimental.pallas import tpu_sc as plsc


_BR = 1000


def _linear_body(x_ref, w_ref, b_ref, o_ref):
    o_ref[...] = lax.dot_general(
        x_ref[...], w_ref[...], (((1,), (1,)), ((), ())),
        preferred_element_type=jnp.float32) + b_ref[...]


def _tc_linear(x, w, b):
    n, d = x.shape
    return pl.pallas_call(
        _linear_body,
        grid=(n // _BR,),
        in_specs=[
            pl.BlockSpec((_BR, d), lambda i: (i, 0)),
            pl.BlockSpec((d, d), lambda i: (0, 0)),
            pl.BlockSpec((1, d), lambda i: (0, 0)),
        ],
        out_specs=pl.BlockSpec((_BR, d), lambda i: (i, 0)),
        out_shape=jax.ShapeDtypeStruct((n, d), jnp.float32),
    )(x, w, b.reshape(1, d))


def _combine_body(p_ref, h_ref, wrel_ref, brel_ref, wroot_ref, o_ref):
    agg = p_ref[0]
    acc = lax.dot_general(agg, wrel_ref[...], (((1,), (1,)), ((), ())),
                          preferred_element_type=jnp.float32)
    acc += lax.dot_general(h_ref[...], wroot_ref[...], (((1,), (1,)), ((), ())),
                           preferred_element_type=jnp.float32)
    o_ref[...] = jnp.maximum(acc + brel_ref[...], 0.0)


def _tc_combine(p, h, w_rel, b_rel, w_root):
    n, d = h.shape
    hb = (n // 2) // _BR
    return pl.pallas_call(
        _combine_body,
        grid=(n // _BR,),
        in_specs=[
            pl.BlockSpec((1, _BR, d), lambda i: (i // hb, i % hb, 0)),
            pl.BlockSpec((_BR, d), lambda i: (i, 0)),
            pl.BlockSpec((d, d), lambda i: (0, 0)),
            pl.BlockSpec((1, d), lambda i: (0, 0)),
            pl.BlockSpec((d, d), lambda i: (0, 0)),
        ],
        out_specs=pl.BlockSpec((_BR, d), lambda i: (i, 0)),
        out_shape=jax.ShapeDtypeStruct((n, d), jnp.float32),
    )(p, h, w_rel, b_rel.reshape(1, d), w_root)


_B = 64
_CB = 8
_CE = _CB * _B
_WBR = 88


def _nacc(n):
    return ((n // 2 + 16) + _WBR - 1) // _WBR * _WBR


def _halves(e_per_w):
    h0 = (e_per_w // 2 + 255) // 256 * 256
    return (h0, e_per_w - h0)


def _in_chunks(sz):
    out = []
    while sz > 0:
        c = min(sz, 1280)
        out.append(c)
        sz -= c
    return out


def _partition_body(ns, e_per_w, half, cap,
                    src_hbm, dst_hbm, lsrc_hbm, ldst_hbm, cnt_hbm,
                    insrc_v, indst_v, os0_v, od0_v, os1_v, od1_v, cnt_v):
    c = lax.axis_index("c")
    s = lax.axis_index("s")
    wid = c * ns + s
    wbase = pl.multiple_of(wid * e_per_w, e_per_w)
    iota = lax.iota(jnp.int32, 16)
    dummy = half + lax.rem(wid, 16)
    outs = ((os0_v, od0_v), (os1_v, od1_v))
    trash = cap * _CE

    def scatter_pos(pos):
        i0 = lax.shift_right_arithmetic(pos, 9)
        r = lax.bitwise_and(pos, _CE - 1)
        i1 = lax.shift_right_arithmetic(r, 6)
        return i0, i1, lax.bitwise_and(r, _B - 1)

    cv = iota * 0
    hoff = 0
    for h, hsz in enumerate(_halves(e_per_w)):
        counts = (jnp.int32(0), jnp.int32(0))
        coff = 0
        for csz in _in_chunks(hsz):
            off = pl.multiple_of(wbase + hoff + coff, 8)
            pltpu.sync_copy(src_hbm.at[pl.ds(off, csz)],
                            insrc_v.at[pl.ds(0, csz)])
            pltpu.sync_copy(dst_hbm.at[pl.ds(off, csz)],
                            indst_v.at[pl.ds(0, csz)])

            def group(g, carry):
                o = pl.multiple_of(g * 16, 16)
                sv = insrc_v[pl.ds(o, 16)]
                dv = indst_v[pl.ds(o, 16)]
                new = []
                for r in range(2):
                    t = dv - r * half
                    neg = lax.shift_right_logical(t, 31)
                    over = lax.shift_right_logical(half - 1 - t, 31)
                    mi = 1 - lax.bitwise_or(neg, over)
                    pc = plsc.cumsum(mi)
                    pos = carry[r] + pc - 1
                    pos = pos * mi + trash * (1 - mi)
                    i0, i1, i2 = scatter_pos(pos)
                    plsc.store_scatter(outs[r][0], [i0, i1, i2], sv)
                    plsc.store_scatter(outs[r][1], [i0, i1, i2], t)
                    new.append(carry[r] + pc[15])
                return tuple(new)

            counts = lax.fori_loop(0, csz // 16, group, counts)
            coff += csz
        hoff += hsz

        for r in range(2):
            cnt = counts[r]
            padded = lax.shift_left(
                lax.shift_right_arithmetic(cnt + _CE - 1, 9), 9)
            for k in range(_CE // 16):
                pos = cnt + k * 16 + iota
                mi = 1 - lax.shift_right_logical(padded - 1 - pos, 31)
                pos = pos * mi + trash * (1 - mi)
                i0, i1, i2 = scatter_pos(pos)
                plsc.store_scatter(outs[r][0], [i0, i1, i2], iota * 0)
                plsc.store_scatter(outs[r][1], [i0, i1, i2],
                                   iota * 0 + dummy)
            nch = lax.shift_right_arithmetic(padded, 9)
            lane = 1 - jnp.minimum(jnp.abs(iota - (4 * h + r)), 1)
            cv = cv + lane * nch

        for r in range(2):
            pltpu.sync_copy(outs[r][0].at[pl.ds(0, cap)],
                            lsrc_hbm.at[r, wid, h])
            pltpu.sync_copy(outs[r][1].at[pl.ds(0, cap)],
                            ldst_hbm.at[r, wid, h])

    cnt_v[pl.ds(0, 16)] = cv
    pltpu.sync_copy(cnt_v, cnt_hbm.at[wid])


def _sc_partition(src, dst, n):
    e = src.shape[0]
    info = plsc.get_sparse_core_info()
    ns = info.num_subcores
    nw = 2 * ns
    e_per_w = e // nw
    assert e == nw * e_per_w and e_per_w % 16 == 0
    h0, h1 = _halves(e_per_w)
    assert h1 > 0 and h1 % 16 == 0
    cap = (max(h0, h1) + _CE - 1) // _CE
    mesh = plsc.VectorSubcoreMesh(core_axis_name="c", subcore_axis_name="s")
    kern = pl.kernel(
        functools.partial(_partition_body, ns, e_per_w, n // 2, cap),
        compiler_params=pltpu.CompilerParams(needs_layout_passes=False),
        out_type=[
            jax.ShapeDtypeStruct((2, nw, 2, cap, _CB, _B), jnp.int32),
            jax.ShapeDtypeStruct((2, nw, 2, cap, _CB, _B), jnp.int32),
            jax.ShapeDtypeStruct((nw, 16), jnp.int32),
        ],
        mesh=mesh,
        scratch_types=[
            pltpu.VMEM((1280,), jnp.int32),
            pltpu.VMEM((1280,), jnp.int32),
            pltpu.VMEM((cap + 1, _CB, _B), jnp.int32),
            pltpu.VMEM((cap + 1, _CB, _B), jnp.int32),
            pltpu.VMEM((cap + 1, _CB, _B), jnp.int32),
            pltpu.VMEM((cap + 1, _CB, _B), jnp.int32),
            pltpu.VMEM((16,), jnp.int32),
        ],
    )
    return kern(src, dst)


def _segsum_body2(ns, n,
                  h_hbm, lsrc_hbm, ldst_hbm, cnt_hbm, out_hbm,
                  cnt0_v, cnt1_v, src_v, dst_v, rows_v, acc_s,
                  sg0, sg1, ss, sc0, sc1):
    c = lax.axis_index("c")
    s = lax.axis_index("s")
    sgs = (sg0, sg1)
    scs = (sc0, sc1)
    nacc = _nacc(n)
    cnts = (cnt0_v, cnt1_v)

    pltpu.sync_copy(cnt_hbm.at[2 * s], cnt0_v)
    pltpu.sync_copy(cnt_hbm.at[2 * s + 1], cnt1_v)

    for i in range(rows_v.shape[1]):
        for k in range(rows_v.shape[2] // 16):
            rows_v[0, i, pl.ds(k * 16, 16)] = jnp.zeros((16,), jnp.float32)
    nwb = nacc // _WBR
    for k in range((nwb + ns - 1) // ns):
        ci = k * ns + s

        @pl.when(ci < nwb)
        def _():
            r0 = ci * _WBR
            for m in range(_WBR // 8):
                pltpu.sync_copy(rows_v.at[0, pl.ds(0, 8)],
                                acc_s.at[pl.ds(r0 + m * 8, 8)])
    plsc.subcore_barrier()

    def process_list(li, h):
        w = 2 * s + li
        cv = cnts[li][pl.ds(0, 16)]
        nch = cv[4 * h] * (1 - c) + cv[4 * h + 1] * c

        def stage(cc, p):
            pltpu.async_copy(lsrc_hbm.at[c, w, h, cc], src_v.at[p], ss)
            pltpu.async_copy(ldst_hbm.at[c, w, h, cc], dst_v.at[p], ss)

        def drain_stage(p):
            pltpu.make_async_copy(lsrc_hbm.at[c, w, h, 0], src_v.at[p],
                                  ss).wait()
            pltpu.make_async_copy(ldst_hbm.at[c, w, h, 0], dst_v.at[p],
                                  ss).wait()

        def process(p):
            def fire(j):
                return pltpu.async_copy(h_hbm.at[src_v.at[p, j]],
                                        rows_v.at[j % 2], sgs[j % 2])

            gds = [None] * _CB
            sds = [None] * _CB
            gds[0] = fire(0)
            for j in range(_CB):
                if j >= 1:
                    sds[j - 1].wait()
                if j + 1 < _CB:
                    gds[j + 1] = fire(j + 1)
                gds[j].wait()
                sds[j] = pltpu.async_copy(rows_v.at[j % 2],
                                          acc_s.at[dst_v.at[p, j]],
                                          scs[j % 2], add=True)
            sds[_CB - 1].wait()

        @pl.when(nch > 0)
        def _():
            stage(0, 0)

        def body(ci, carry):
            p = lax.rem(ci, 2)
            drain_stage(p)

            @pl.when(ci + 1 < nch)
            def _():
                stage(ci + 1, 1 - p)
            process(p)
            return carry

        lax.fori_loop(0, nch, body, 0)

    for li in range(2):
        for h in range(2):
            process_list(li, h)
    plsc.subcore_barrier()

    nwb = nacc // _WBR
    for k in range((nwb + ns - 1) // ns):
        ci = k * ns + s

        @pl.when(ci < nwb)
        def _():
            r0 = ci * _WBR
            pltpu.sync_copy(acc_s.at[pl.ds(r0, _WBR)],
                            out_hbm.at[c, pl.ds(r0, _WBR)])


def _sc_segment_sum(h, lsrc, ldst, cnt):
    n, d = h.shape
    info = plsc.get_sparse_core_info()
    ns = info.num_subcores
    nacc = _nacc(n)
    mesh = plsc.VectorSubcoreMesh(core_axis_name="c", subcore_axis_name="s")
    kern = pl.kernel(
        functools.partial(_segsum_body2, ns, n),
        out_type=jax.ShapeDtypeStruct((2, nacc, d), jnp.float32),
        mesh=mesh,
        scratch_types=[
            pltpu.VMEM((16,), jnp.int32),
            pltpu.VMEM((16,), jnp.int32),
            pltpu.VMEM((2, _CB, _B), jnp.int32),
            pltpu.VMEM((2, _CB, _B), jnp.int32),
            pltpu.VMEM((2, _B, d), jnp.float32),
            pltpu.VMEM_SHARED((nacc, d), jnp.float32),
            pltpu.SemaphoreType.DMA,
            pltpu.SemaphoreType.DMA,
            pltpu.SemaphoreType.DMA,
            pltpu.SemaphoreType.DMA,
            pltpu.SemaphoreType.DMA,
        ],
    )
    return kern(h, lsrc, ldst, cnt)


def kernel(x, edge_index, W_in, b_in, W_rel1, b_rel1, W_root1,
           W_rel2, b_rel2, W_root2):
    n, _ = x.shape
    assert n % 2 == 0 and (n // 2) % _BR == 0

    lsrc, ldst, cnt = _sc_partition(edge_index[0], edge_index[1], n)

    h0 = _tc_linear(x, W_in, b_in)
    p1 = _sc_segment_sum(h0, lsrc, ldst, cnt)
    h1 = _tc_combine(p1, h0, W_rel1, b_rel1, W_root1)
    p2 = _sc_segment_sum(h1, lsrc, ldst, cnt)
    h2 = _tc_combine(p2, h1, W_rel2, b_rel2, W_root2)
    return h2


# speedup vs baseline: 1.0001x; 1.0001x over previous
import functools

import jax
import jax.numpy as jnp
from jax import lax
from jax.experimental import pallas as pl
from jax.experimental.pallas import tpu as pltpu
from jax.experimental.pallas import tpu_sc as plsc


_BR = 1000


def _linear_body(x_ref, w_ref, b_ref, o_ref):
    o_ref[...] = lax.dot_general(
        x_ref[...], w_ref[...], (((1,), (1,)), ((), ())),
        preferred_element_type=jnp.float32) + b_ref[...]


def _tc_linear(x, w, b):
    n, d = x.shape
    return pl.pallas_call(
        _linear_body,
        grid=(n // _BR,),
        in_specs=[
            pl.BlockSpec((_BR, d), lambda i: (i, 0)),
            pl.BlockSpec((d, d), lambda i: (0, 0)),
            pl.BlockSpec((1, d), lambda i: (0, 0)),
        ],
        out_specs=pl.BlockSpec((_BR, d), lambda i: (i, 0)),
        out_shape=jax.ShapeDtypeStruct((n, d), jnp.float32),
    )(x, w, b.reshape(1, d))


def _combine_body(p_ref, h_ref, wrel_ref, brel_ref, wroot_ref, o_ref):
    agg = p_ref[0]
    acc = lax.dot_general(agg, wrel_ref[...], (((1,), (1,)), ((), ())),
                          preferred_element_type=jnp.float32)
    acc += lax.dot_general(h_ref[...], wroot_ref[...], (((1,), (1,)), ((), ())),
                           preferred_element_type=jnp.float32)
    o_ref[...] = jnp.maximum(acc + brel_ref[...], 0.0)


def _tc_combine(p, h, w_rel, b_rel, w_root):
    n, d = h.shape
    hb = (n // 2) // _BR
    return pl.pallas_call(
        _combine_body,
        grid=(n // _BR,),
        in_specs=[
            pl.BlockSpec((1, _BR, d), lambda i: (i // hb, i % hb, 0)),
            pl.BlockSpec((_BR, d), lambda i: (i, 0)),
            pl.BlockSpec((d, d), lambda i: (0, 0)),
            pl.BlockSpec((1, d), lambda i: (0, 0)),
            pl.BlockSpec((d, d), lambda i: (0, 0)),
        ],
        out_specs=pl.BlockSpec((_BR, d), lambda i: (i, 0)),
        out_shape=jax.ShapeDtypeStruct((n, d), jnp.float32),
    )(p, h, w_rel, b_rel.reshape(1, d), w_root)


_B = 64
_CB = 8
_CE = _CB * _B
_WBR = 88


def _nacc(n):
    return ((n // 2 + 16) + _WBR - 1) // _WBR * _WBR


def _halves(e_per_w):
    h0 = (e_per_w // 2 + 255) // 256 * 256
    return (h0, e_per_w - h0)


def _in_chunks(sz):
    out = []
    while sz > 0:
        c = min(sz, 1280)
        out.append(c)
        sz -= c
    return out


def _partition_body(ns, e_per_w, half, cap,
                    src_hbm, dst_hbm, lsrc_hbm, ldst_hbm, cnt_hbm,
                    insrc_v, indst_v, os0_v, od0_v, os1_v, od1_v, cnt_v):
    c = lax.axis_index("c")
    s = lax.axis_index("s")
    wid = c * ns + s
    wbase = pl.multiple_of(wid * e_per_w, e_per_w)
    iota = lax.iota(jnp.int32, 16)
    dummy = half + lax.rem(wid, 16)
    outs = ((os0_v, od0_v), (os1_v, od1_v))
    trash = cap * _CE

    def scatter_pos(pos):
        i0 = lax.shift_right_arithmetic(pos, 9)
        r = lax.bitwise_and(pos, _CE - 1)
        i1 = lax.shift_right_arithmetic(r, 6)
        return i0, i1, lax.bitwise_and(r, _B - 1)

    cv = iota * 0
    hoff = 0
    for h, hsz in enumerate(_halves(e_per_w)):
        counts = (jnp.int32(0), jnp.int32(0))
        coff = 0
        for csz in _in_chunks(hsz):
            off = pl.multiple_of(wbase + hoff + coff, 8)
            pltpu.sync_copy(src_hbm.at[pl.ds(off, csz)],
                            insrc_v.at[pl.ds(0, csz)])
            pltpu.sync_copy(dst_hbm.at[pl.ds(off, csz)],
                            indst_v.at[pl.ds(0, csz)])

            def group(g, carry):
                o = pl.multiple_of(g * 16, 16)
                sv = insrc_v[pl.ds(o, 16)]
                dv = indst_v[pl.ds(o, 16)]
                new = []
                for r in range(2):
                    t = dv - r * half
                    neg = lax.shift_right_logical(t, 31)
                    over = lax.shift_right_logical(half - 1 - t, 31)
                    mi = 1 - lax.bitwise_or(neg, over)
                    pc = plsc.cumsum(mi)
                    pos = carry[r] + pc - 1
                    pos = pos * mi + trash * (1 - mi)
                    i0, i1, i2 = scatter_pos(pos)
                    plsc.store_scatter(outs[r][0], [i0, i1, i2], sv)
                    plsc.store_scatter(outs[r][1], [i0, i1, i2], t)
                    new.append(carry[r] + pc[15])
                return tuple(new)

            counts = lax.fori_loop(0, csz // 16, group, counts)
            coff += csz
        hoff += hsz

        for r in range(2):
            cnt = counts[r]
            padded = lax.shift_left(
                lax.shift_right_arithmetic(cnt + _CE - 1, 9), 9)
            for k in range(_CE // 16):
                pos = cnt + k * 16 + iota
                mi = 1 - lax.shift_right_logical(padded - 1 - pos, 31)
                pos = pos * mi + trash * (1 - mi)
                i0, i1, i2 = scatter_pos(pos)
                plsc.store_scatter(outs[r][0], [i0, i1, i2], iota * 0)
                plsc.store_scatter(outs[r][1], [i0, i1, i2],
                                   iota * 0 + dummy)
            nch = lax.shift_right_arithmetic(padded, 9)
            lane = 1 - jnp.minimum(jnp.abs(iota - (4 * h + r)), 1)
            cv = cv + lane * nch

        for r in range(2):
            pltpu.sync_copy(outs[r][0].at[pl.ds(0, cap)],
                            lsrc_hbm.at[r, wid, h])
            pltpu.sync_copy(outs[r][1].at[pl.ds(0, cap)],
                            ldst_hbm.at[r, wid, h])

    cnt_v[pl.ds(0, 16)] = cv
    pltpu.sync_copy(cnt_v, cnt_hbm.at[wid])


def _sc_partition(src, dst, n):
    e = src.shape[0]
    info = plsc.get_sparse_core_info()
    ns = info.num_subcores
    nw = 2 * ns
    e_per_w = e // nw
    assert e == nw * e_per_w and e_per_w % 16 == 0
    h0, h1 = _halves(e_per_w)
    assert h1 > 0 and h1 % 16 == 0
    cap = (max(h0, h1) + _CE - 1) // _CE
    mesh = plsc.VectorSubcoreMesh(core_axis_name="c", subcore_axis_name="s")
    kern = pl.kernel(
        functools.partial(_partition_body, ns, e_per_w, n // 2, cap),
        compiler_params=pltpu.CompilerParams(needs_layout_passes=False),
        out_type=[
            jax.ShapeDtypeStruct((2, nw, 2, cap, _CB, _B), jnp.int32),
            jax.ShapeDtypeStruct((2, nw, 2, cap, _CB, _B), jnp.int32),
            jax.ShapeDtypeStruct((nw, 16), jnp.int32),
        ],
        mesh=mesh,
        scratch_types=[
            pltpu.VMEM((1280,), jnp.int32),
            pltpu.VMEM((1280,), jnp.int32),
            pltpu.VMEM((cap + 1, _CB, _B), jnp.int32),
            pltpu.VMEM((cap + 1, _CB, _B), jnp.int32),
            pltpu.VMEM((cap + 1, _CB, _B), jnp.int32),
            pltpu.VMEM((cap + 1, _CB, _B), jnp.int32),
            pltpu.VMEM((16,), jnp.int32),
        ],
    )
    return kern(src, dst)


def _segsum_body2(ns, n,
                  h_hbm, lsrc_hbm, ldst_hbm, cnt_hbm, out_hbm,
                  cnt0_v, cnt1_v, src_v, dst_v, rows_v, acc_s,
                  sg0, sg1, ss, sc0, sc1):
    c = lax.axis_index("c")
    s = lax.axis_index("s")
    sgs = (sg0, sg1)
    scs = (sc0, sc1)
    nacc = _nacc(n)
    cnts = (cnt0_v, cnt1_v)
    cap = lsrc_hbm.shape[3]

    pltpu.sync_copy(cnt_hbm.at[2 * s], cnt0_v)
    pltpu.sync_copy(cnt_hbm.at[2 * s + 1], cnt1_v)

    for i in range(rows_v.shape[1]):
        for k in range(rows_v.shape[2] // 16):
            rows_v[0, i, pl.ds(k * 16, 16)] = jnp.zeros((16,), jnp.float32)
    nwb = nacc // _WBR
    for k in range((nwb + ns - 1) // ns):
        ci = k * ns + s

        @pl.when(ci < nwb)
        def _():
            r0 = ci * _WBR
            for m in range(_WBR // 8):
                pltpu.sync_copy(rows_v.at[0, pl.ds(0, 8)],
                                acc_s.at[pl.ds(r0 + m * 8, 8)])
    plsc.subcore_barrier()

    def process_list(li, h):
        w = 2 * s + li
        cv = cnts[li][pl.ds(0, 16)]
        nch = cv[4 * h] * (1 - c) + cv[4 * h + 1] * c

        def stage(cc, p):
            pltpu.async_copy(lsrc_hbm.at[c, w, h, cc], src_v.at[p], ss)
            pltpu.async_copy(ldst_hbm.at[c, w, h, cc], dst_v.at[p], ss)

        def drain_stage(p):
            pltpu.make_async_copy(lsrc_hbm.at[c, w, h, 0], src_v.at[p],
                                  ss).wait()
            pltpu.make_async_copy(ldst_hbm.at[c, w, h, 0], dst_v.at[p],
                                  ss).wait()

        def process(p):
            def fire(j):
                return pltpu.async_copy(h_hbm.at[src_v.at[p, j]],
                                        rows_v.at[j % 2], sgs[j % 2])

            gds = [None] * _CB
            sds = [None] * _CB
            gds[0] = fire(0)
            for j in range(_CB):
                if j >= 1:
                    sds[j - 1].wait()
                if j + 1 < _CB:
                    gds[j + 1] = fire(j + 1)
                gds[j].wait()
                sds[j] = pltpu.async_copy(rows_v.at[j % 2],
                                          acc_s.at[dst_v.at[p, j]],
                                          scs[j % 2], add=True)
            sds[_CB - 1].wait()

        @pl.when(nch > 0)
        def _():
            stage(0, 0)

        def body(ci, carry):
            p = lax.rem(ci, 2)

            @pl.when(ci < nch)
            def _():
                drain_stage(p)

                @pl.when(ci + 1 < nch)
                def _():
                    stage(ci + 1, 1 - p)
                process(p)
            return carry

        lax.fori_loop(0, cap, body, 0)

    for li in range(2):
        for h in range(2):
            process_list(li, h)
    plsc.subcore_barrier()

    nwb = nacc // _WBR
    for k in range((nwb + ns - 1) // ns):
        ci = k * ns + s

        @pl.when(ci < nwb)
        def _():
            r0 = ci * _WBR
            pltpu.sync_copy(acc_s.at[pl.ds(r0, _WBR)],
                            out_hbm.at[c, pl.ds(r0, _WBR)])


def _sc_segment_sum(h, lsrc, ldst, cnt):
    n, d = h.shape
    info = plsc.get_sparse_core_info()
    ns = info.num_subcores
    nacc = _nacc(n)
    mesh = plsc.VectorSubcoreMesh(core_axis_name="c", subcore_axis_name="s")
    kern = pl.kernel(
        functools.partial(_segsum_body2, ns, n),
        out_type=jax.ShapeDtypeStruct((2, nacc, d), jnp.float32),
        mesh=mesh,
        scratch_types=[
            pltpu.VMEM((16,), jnp.int32),
            pltpu.VMEM((16,), jnp.int32),
            pltpu.VMEM((2, _CB, _B), jnp.int32),
            pltpu.VMEM((2, _CB, _B), jnp.int32),
            pltpu.VMEM((2, _B, d), jnp.float32),
            pltpu.VMEM_SHARED((nacc, d), jnp.float32),
            pltpu.SemaphoreType.DMA,
            pltpu.SemaphoreType.DMA,
            pltpu.SemaphoreType.DMA,
            pltpu.SemaphoreType.DMA,
            pltpu.SemaphoreType.DMA,
        ],
    )
    return kern(h, lsrc, ldst, cnt)


def kernel(x, edge_index, W_in, b_in, W_rel1, b_rel1, W_root1,
           W_rel2, b_rel2, W_root2):
    n, _ = x.shape
    assert n % 2 == 0 and (n // 2) % _BR == 0

    lsrc, ldst, cnt = _sc_partition(edge_index[0], edge_index[1], n)

    h0 = _tc_linear(x, W_in, b_in)
    p1 = _sc_segment_sum(h0, lsrc, ldst, cnt)
    h1 = _tc_combine(p1, h0, W_rel1, b_rel1, W_root1)
    p2 = _sc_segment_sum(h1, lsrc, ldst, cnt)
    h2 = _tc_combine(p2, h1, W_rel2, b_rel2, W_root2)
    return h2


# speedup vs baseline: 4.2049x; 4.2047x over previous
import functools

import jax
import jax.numpy as jnp
from jax import lax
from jax.experimental import pallas as pl
from jax.experimental.pallas import tpu as pltpu
from jax.experimental.pallas import tpu_sc as plsc


_BR = 1000


def _linear_body(x_ref, w_ref, b_ref, o_ref):
    o_ref[...] = lax.dot_general(
        x_ref[...], w_ref[...], (((1,), (1,)), ((), ())),
        preferred_element_type=jnp.float32) + b_ref[...]


def _tc_linear(x, w, b):
    n, d = x.shape
    return pl.pallas_call(
        _linear_body,
        grid=(n // _BR,),
        in_specs=[
            pl.BlockSpec((_BR, d), lambda i: (i, 0)),
            pl.BlockSpec((d, d), lambda i: (0, 0)),
            pl.BlockSpec((1, d), lambda i: (0, 0)),
        ],
        out_specs=pl.BlockSpec((_BR, d), lambda i: (i, 0)),
        out_shape=jax.ShapeDtypeStruct((n, d), jnp.float32),
    )(x, w, b.reshape(1, d))


def _combine_body(nps, p_refs_and_rest):
    p_refs = p_refs_and_rest[:nps]
    h_ref, wrel_ref, brel_ref, wroot_ref, o_ref = p_refs_and_rest[nps:]
    agg = p_refs[0][0]
    for pr in p_refs[1:]:
        agg = agg + pr[0]
    acc = lax.dot_general(agg, wrel_ref[...], (((1,), (1,)), ((), ())),
                          preferred_element_type=jnp.float32)
    acc += lax.dot_general(h_ref[...], wroot_ref[...], (((1,), (1,)), ((), ())),
                           preferred_element_type=jnp.float32)
    o_ref[...] = jnp.maximum(acc + brel_ref[...], 0.0)


def _tc_combine(ps, h, w_rel, b_rel, w_root):
    n, d = h.shape
    return pl.pallas_call(
        lambda *refs: _combine_body(len(ps), refs),
        grid=(n // _BR,),
        in_specs=[pl.BlockSpec((1, _BR, d), lambda i: (0, i, 0))
                  for _ in ps] + [
            pl.BlockSpec((_BR, d), lambda i: (i, 0)),
            pl.BlockSpec((d, d), lambda i: (0, 0)),
            pl.BlockSpec((1, d), lambda i: (0, 0)),
            pl.BlockSpec((d, d), lambda i: (0, 0)),
        ],
        out_specs=pl.BlockSpec((_BR, d), lambda i: (i, 0)),
        out_shape=jax.ShapeDtypeStruct((n, d), jnp.float32),
    )(*ps, h, w_rel, b_rel.reshape(1, d), w_root)


_B = 80
_CB = 25
_ZR = 16
_WB = 80


def _sc_segsum_body(ns, nchunks, n,
                    h_hbm, src_hbm, dst_hbm, out_hbm,
                    src_v, dst_v, rows_v, zb_v, acc_s,
                    sg0, sg1, ss, sc0, sc1):
    c = lax.axis_index("c")
    s = lax.axis_index("s")
    wid = c * ns + s
    sgs = (sg0, sg1)
    scs = (sc0, sc1)

    for i in range(_ZR):
        for k in range(zb_v.shape[1] // 16):
            zb_v[i, pl.ds(k * 16, 16)] = jnp.zeros((16,), jnp.float32)
    nwb = n // _WB
    for k in range((nwb + ns - 1) // ns):
        ci = k * ns + s

        @pl.when(ci < nwb)
        def _():
            r0 = ci * _WB
            for m in range(_WB // _ZR):
                pltpu.sync_copy(zb_v, acc_s.at[pl.ds(r0 + m * _ZR, _ZR)])
    plsc.subcore_barrier()

    def stage(cc, p):
        pltpu.async_copy(src_hbm.at[wid, cc], src_v.at[p], ss)
        pltpu.async_copy(dst_hbm.at[wid, cc], dst_v.at[p], ss)

    def drain_stage(p):
        pltpu.make_async_copy(src_hbm.at[wid, 0], src_v.at[p], ss).wait()
        pltpu.make_async_copy(dst_hbm.at[wid, 0], dst_v.at[p], ss).wait()

    def process(p):
        def fire(j):
            return pltpu.async_copy(h_hbm.at[src_v.at[p, j]],
                                    rows_v.at[j % 2], sgs[j % 2])

        gds = [None] * _CB
        sds = [None] * _CB
        gds[0] = fire(0)
        for j in range(_CB):
            if j >= 1:
                sds[j - 1].wait()
            if j + 1 < _CB:
                gds[j + 1] = fire(j + 1)
            gds[j].wait()
            sds[j] = pltpu.async_copy(rows_v.at[j % 2],
                                      acc_s.at[dst_v.at[p, j]],
                                      scs[j % 2], add=True)
        sds[_CB - 1].wait()

    stage(0, 0)

    def body(ci, carry):
        p = lax.rem(ci, 2)
        drain_stage(p)

        @pl.when(ci + 1 < nchunks)
        def _():
            stage(ci + 1, 1 - p)
        process(p)
        return carry

    lax.fori_loop(0, nchunks, body, 0)
    plsc.subcore_barrier()

    for k in range((nwb + ns - 1) // ns):
        ci = k * ns + s

        @pl.when(ci < nwb)
        def _():
            r0 = ci * _WB
            pltpu.sync_copy(acc_s.at[pl.ds(r0, _WB)],
                            out_hbm.at[c, pl.ds(r0, _WB)])


def _sc_segment_sum(h, src4, dst4):
    n, d = h.shape
    nw, nchunks, cb, b = dst4.shape
    info = plsc.get_sparse_core_info()
    ns = info.num_subcores
    assert nw == ns and cb == _CB and b == _B
    assert n % _WB == 0 and _WB % _ZR == 0
    mesh = plsc.VectorSubcoreMesh(core_axis_name="c", subcore_axis_name="s",
                                  num_cores=1)
    kern = pl.kernel(
        functools.partial(_sc_segsum_body, ns, nchunks, n),
        out_type=jax.ShapeDtypeStruct((1, n, d), jnp.float32),
        mesh=mesh,
        scratch_types=[
            pltpu.VMEM((2, _CB, _B), jnp.int32),
            pltpu.VMEM((2, _CB, _B), jnp.int32),
            pltpu.VMEM((2, _B, d), jnp.float32),
            pltpu.VMEM((_ZR, d), jnp.float32),
            pltpu.VMEM_SHARED((n, d), jnp.float32),
            pltpu.SemaphoreType.DMA,
            pltpu.SemaphoreType.DMA,
            pltpu.SemaphoreType.DMA,
            pltpu.SemaphoreType.DMA,
            pltpu.SemaphoreType.DMA,
        ],
    )
    return kern(h, src4, dst4)


def kernel(x, edge_index, W_in, b_in, W_rel1, b_rel1, W_root1,
           W_rel2, b_rel2, W_root2):
    e = edge_index.shape[1]
    info = plsc.get_sparse_core_info()
    ns = info.num_subcores
    cedges = _CB * _B
    assert e % (ns * cedges) == 0
    nchunks = e // (ns * cedges)

    src4 = edge_index[0].reshape(ns, nchunks, _CB, _B)
    dst4 = edge_index[1].reshape(ns, nchunks, _CB, _B)

    h0 = _tc_linear(x, W_in, b_in)
    p1 = _sc_segment_sum(h0, src4, dst4)
    h1 = _tc_combine([p1], h0, W_rel1, b_rel1, W_root1)
    p2 = _sc_segment_sum(h1, src4, dst4)
    h2 = _tc_combine([p2], h1, W_rel2, b_rel2, W_root2)
    return h2


# speedup vs baseline: 4.4861x; 1.0669x over previous
import functools

import jax
import jax.numpy as jnp
from jax import lax
from jax.experimental import pallas as pl
from jax.experimental.pallas import tpu as pltpu
from jax.experimental.pallas import tpu_sc as plsc


_BR = 1000


def _linear_body(x_ref, w_ref, b_ref, o_ref):
    o_ref[...] = lax.dot_general(
        x_ref[...], w_ref[...], (((1,), (1,)), ((), ())),
        preferred_element_type=jnp.float32) + b_ref[...]


def _tc_linear(x, w, b):
    n, d = x.shape
    return pl.pallas_call(
        _linear_body,
        grid=(n // _BR,),
        in_specs=[
            pl.BlockSpec((_BR, d), lambda i: (i, 0)),
            pl.BlockSpec((d, d), lambda i: (0, 0)),
            pl.BlockSpec((1, d), lambda i: (0, 0)),
        ],
        out_specs=pl.BlockSpec((_BR, d), lambda i: (i, 0)),
        out_shape=jax.ShapeDtypeStruct((n, d), jnp.float32),
    )(x, w, b.reshape(1, d))


def _combine_body(nps, p_refs_and_rest):
    p_refs = p_refs_and_rest[:nps]
    h_ref, wrel_ref, brel_ref, wroot_ref, o_ref = p_refs_and_rest[nps:]
    agg = p_refs[0][0]
    for pr in p_refs[1:]:
        agg = agg + pr[0]
    acc = lax.dot_general(agg, wrel_ref[...], (((1,), (1,)), ((), ())),
                          preferred_element_type=jnp.float32)
    acc += lax.dot_general(h_ref[...], wroot_ref[...], (((1,), (1,)), ((), ())),
                           preferred_element_type=jnp.float32)
    o_ref[...] = jnp.maximum(acc + brel_ref[...], 0.0)


def _tc_combine(ps, h, w_rel, b_rel, w_root):
    n, d = h.shape
    return pl.pallas_call(
        lambda *refs: _combine_body(len(ps), refs),
        grid=(n // _BR,),
        in_specs=[pl.BlockSpec((1, _BR, d), lambda i: (0, i, 0))
                  for _ in ps] + [
            pl.BlockSpec((_BR, d), lambda i: (i, 0)),
            pl.BlockSpec((d, d), lambda i: (0, 0)),
            pl.BlockSpec((1, d), lambda i: (0, 0)),
            pl.BlockSpec((d, d), lambda i: (0, 0)),
        ],
        out_specs=pl.BlockSpec((_BR, d), lambda i: (i, 0)),
        out_shape=jax.ShapeDtypeStruct((n, d), jnp.float32),
    )(*ps, h, w_rel, b_rel.reshape(1, d), w_root)


_B = 100
_CB = 25
_ZR = 16
_WB = 80


def _sc_segsum_body(ns, nchunks, n,
                    h_hbm, src_hbm, dst_hbm, out_hbm,
                    src_v, dst_v, rows_v, zb_v, acc_s,
                    sg0, sg1, ss, sc0, sc1):
    c = lax.axis_index("c")
    s = lax.axis_index("s")
    wid = c * ns + s
    sgs = (sg0, sg1)
    scs = (sc0, sc1)

    for i in range(_ZR):
        for k in range(zb_v.shape[1] // 16):
            zb_v[i, pl.ds(k * 16, 16)] = jnp.zeros((16,), jnp.float32)
    nwb = n // _WB
    for k in range((nwb + ns - 1) // ns):
        ci = k * ns + s

        @pl.when(ci < nwb)
        def _():
            r0 = ci * _WB
            for m in range(_WB // _ZR):
                pltpu.sync_copy(zb_v, acc_s.at[pl.ds(r0 + m * _ZR, _ZR)])
    plsc.subcore_barrier()

    def stage(cc, p):
        pltpu.async_copy(src_hbm.at[wid, cc], src_v.at[p], ss)
        pltpu.async_copy(dst_hbm.at[wid, cc], dst_v.at[p], ss)

    def drain_stage(p):
        pltpu.make_async_copy(src_hbm.at[wid, 0], src_v.at[p], ss).wait()
        pltpu.make_async_copy(dst_hbm.at[wid, 0], dst_v.at[p], ss).wait()

    def process(p):
        def fire(j):
            return pltpu.async_copy(h_hbm.at[src_v.at[p, j]],
                                    rows_v.at[j % 2], sgs[j % 2])

        gds = [None] * _CB
        sds = [None] * _CB
        gds[0] = fire(0)
        for j in range(_CB):
            if j >= 1:
                sds[j - 1].wait()
            if j + 1 < _CB:
                gds[j + 1] = fire(j + 1)
            gds[j].wait()
            sds[j] = pltpu.async_copy(rows_v.at[j % 2],
                                      acc_s.at[dst_v.at[p, j]],
                                      scs[j % 2], add=True)
        sds[_CB - 1].wait()

    stage(0, 0)

    def body(ci, carry):
        p = lax.rem(ci, 2)
        drain_stage(p)

        @pl.when(ci + 1 < nchunks)
        def _():
            stage(ci + 1, 1 - p)
        process(p)
        return carry

    lax.fori_loop(0, nchunks, body, 0)
    plsc.subcore_barrier()

    for k in range((nwb + ns - 1) // ns):
        ci = k * ns + s

        @pl.when(ci < nwb)
        def _():
            r0 = ci * _WB
            pltpu.sync_copy(acc_s.at[pl.ds(r0, _WB)],
                            out_hbm.at[c, pl.ds(r0, _WB)])


def _sc_segment_sum(h, src4, dst4):
    n, d = h.shape
    nw, nchunks, cb, b = dst4.shape
    info = plsc.get_sparse_core_info()
    ns = info.num_subcores
    assert nw == ns and cb == _CB and b == _B
    assert n % _WB == 0 and _WB % _ZR == 0
    mesh = plsc.VectorSubcoreMesh(core_axis_name="c", subcore_axis_name="s",
                                  num_cores=1)
    kern = pl.kernel(
        functools.partial(_sc_segsum_body, ns, nchunks, n),
        out_type=jax.ShapeDtypeStruct((1, n, d), jnp.float32),
        mesh=mesh,
        scratch_types=[
            pltpu.VMEM((2, _CB, _B), jnp.int32),
            pltpu.VMEM((2, _CB, _B), jnp.int32),
            pltpu.VMEM((2, _B, d), jnp.float32),
            pltpu.VMEM((_ZR, d), jnp.float32),
            pltpu.VMEM_SHARED((n, d), jnp.float32),
            pltpu.SemaphoreType.DMA,
            pltpu.SemaphoreType.DMA,
            pltpu.SemaphoreType.DMA,
            pltpu.SemaphoreType.DMA,
            pltpu.SemaphoreType.DMA,
        ],
    )
    return kern(h, src4, dst4)


def kernel(x, edge_index, W_in, b_in, W_rel1, b_rel1, W_root1,
           W_rel2, b_rel2, W_root2):
    e = edge_index.shape[1]
    info = plsc.get_sparse_core_info()
    ns = info.num_subcores
    cedges = _CB * _B
    assert e % (ns * cedges) == 0
    nchunks = e // (ns * cedges)

    src4 = edge_index[0].reshape(ns, nchunks, _CB, _B)
    dst4 = edge_index[1].reshape(ns, nchunks, _CB, _B)

    h0 = _tc_linear(x, W_in, b_in)
    p1 = _sc_segment_sum(h0, src4, dst4)
    h1 = _tc_combine([p1], h0, W_rel1, b_rel1, W_root1)
    p2 = _sc_segment_sum(h1, src4, dst4)
    h2 = _tc_combine([p2], h1, W_rel2, b_rel2, W_root2)
    return h2


# speedup vs baseline: 4.6273x; 1.0315x over previous
import functools

import jax
import jax.numpy as jnp
from jax import lax
from jax.experimental import pallas as pl
from jax.experimental.pallas import tpu as pltpu
from jax.experimental.pallas import tpu_sc as plsc


_BR = 1000


def _linear_body(x_ref, w_ref, b_ref, o_ref):
    o_ref[...] = lax.dot_general(
        x_ref[...], w_ref[...], (((1,), (1,)), ((), ())),
        preferred_element_type=jnp.float32) + b_ref[...]


def _tc_linear(x, w, b):
    n, d = x.shape
    return pl.pallas_call(
        _linear_body,
        grid=(n // _BR,),
        in_specs=[
            pl.BlockSpec((_BR, d), lambda i: (i, 0)),
            pl.BlockSpec((d, d), lambda i: (0, 0)),
            pl.BlockSpec((1, d), lambda i: (0, 0)),
        ],
        out_specs=pl.BlockSpec((_BR, d), lambda i: (i, 0)),
        out_shape=jax.ShapeDtypeStruct((n, d), jnp.float32),
    )(x, w, b.reshape(1, d))


def _combine_body(nps, p_refs_and_rest):
    p_refs = p_refs_and_rest[:nps]
    h_ref, wrel_ref, brel_ref, wroot_ref, o_ref = p_refs_and_rest[nps:]
    agg = p_refs[0][0]
    for pr in p_refs[1:]:
        agg = agg + pr[0]
    acc = lax.dot_general(agg, wrel_ref[...], (((1,), (1,)), ((), ())),
                          preferred_element_type=jnp.float32)
    acc += lax.dot_general(h_ref[...], wroot_ref[...], (((1,), (1,)), ((), ())),
                           preferred_element_type=jnp.float32)
    o_ref[...] = jnp.maximum(acc + brel_ref[...], 0.0)


def _tc_combine(ps, h, w_rel, b_rel, w_root):
    n, d = h.shape
    return pl.pallas_call(
        lambda *refs: _combine_body(len(ps), refs),
        grid=(n // _BR,),
        in_specs=[pl.BlockSpec((1, _BR, d), lambda i: (0, i, 0))
                  for _ in ps] + [
            pl.BlockSpec((_BR, d), lambda i: (i, 0)),
            pl.BlockSpec((d, d), lambda i: (0, 0)),
            pl.BlockSpec((1, d), lambda i: (0, 0)),
            pl.BlockSpec((d, d), lambda i: (0, 0)),
        ],
        out_specs=pl.BlockSpec((_BR, d), lambda i: (i, 0)),
        out_shape=jax.ShapeDtypeStruct((n, d), jnp.float32),
    )(*ps, h, w_rel, b_rel.reshape(1, d), w_root)


_B = 125
_CB = 20
_ZR = 16
_WB = 80


def _sc_segsum_body(ns, nchunks, n,
                    h_hbm, src_hbm, dst_hbm, out_hbm,
                    src_v, dst_v, rows_v, zb_v, acc_s,
                    sg0, sg1, ss, sc0, sc1):
    c = lax.axis_index("c")
    s = lax.axis_index("s")
    wid = c * ns + s
    sgs = (sg0, sg1)
    scs = (sc0, sc1)

    for i in range(_ZR):
        for k in range(zb_v.shape[1] // 16):
            zb_v[i, pl.ds(k * 16, 16)] = jnp.zeros((16,), jnp.float32)
    nwb = n // _WB
    for k in range((nwb + ns - 1) // ns):
        ci = k * ns + s

        @pl.when(ci < nwb)
        def _():
            r0 = ci * _WB
            for m in range(_WB // _ZR):
                pltpu.sync_copy(zb_v, acc_s.at[pl.ds(r0 + m * _ZR, _ZR)])
    plsc.subcore_barrier()

    def stage(cc, p):
        pltpu.async_copy(src_hbm.at[wid, cc], src_v.at[p], ss)
        pltpu.async_copy(dst_hbm.at[wid, cc], dst_v.at[p], ss)

    def drain_stage(p):
        pltpu.make_async_copy(src_hbm.at[wid, 0], src_v.at[p], ss).wait()
        pltpu.make_async_copy(dst_hbm.at[wid, 0], dst_v.at[p], ss).wait()

    def process(p):
        def fire(j):
            return pltpu.async_copy(h_hbm.at[src_v.at[p, j]],
                                    rows_v.at[j % 2], sgs[j % 2])

        gds = [None] * _CB
        sds = [None] * _CB
        gds[0] = fire(0)
        for j in range(_CB):
            if j >= 1:
                sds[j - 1].wait()
            if j + 1 < _CB:
                gds[j + 1] = fire(j + 1)
            gds[j].wait()
            sds[j] = pltpu.async_copy(rows_v.at[j % 2],
                                      acc_s.at[dst_v.at[p, j]],
                                      scs[j % 2], add=True)
        sds[_CB - 1].wait()

    stage(0, 0)

    def body(ci, carry):
        p = lax.rem(ci, 2)
        drain_stage(p)

        @pl.when(ci + 1 < nchunks)
        def _():
            stage(ci + 1, 1 - p)
        process(p)
        return carry

    lax.fori_loop(0, nchunks, body, 0)
    plsc.subcore_barrier()

    for k in range((nwb + ns - 1) // ns):
        ci = k * ns + s

        @pl.when(ci < nwb)
        def _():
            r0 = ci * _WB
            pltpu.sync_copy(acc_s.at[pl.ds(r0, _WB)],
                            out_hbm.at[c, pl.ds(r0, _WB)])


def _sc_segment_sum(h, src4, dst4):
    n, d = h.shape
    nw, nchunks, cb, b = dst4.shape
    info = plsc.get_sparse_core_info()
    ns = info.num_subcores
    assert nw == ns and cb == _CB and b == _B
    assert n % _WB == 0 and _WB % _ZR == 0
    mesh = plsc.VectorSubcoreMesh(core_axis_name="c", subcore_axis_name="s",
                                  num_cores=1)
    kern = pl.kernel(
        functools.partial(_sc_segsum_body, ns, nchunks, n),
        out_type=jax.ShapeDtypeStruct((1, n, d), jnp.float32),
        mesh=mesh,
        scratch_types=[
            pltpu.VMEM((2, _CB, _B), jnp.int32),
            pltpu.VMEM((2, _CB, _B), jnp.int32),
            pltpu.VMEM((2, _B, d), jnp.float32),
            pltpu.VMEM((_ZR, d), jnp.float32),
            pltpu.VMEM_SHARED((n, d), jnp.float32),
            pltpu.SemaphoreType.DMA,
            pltpu.SemaphoreType.DMA,
            pltpu.SemaphoreType.DMA,
            pltpu.SemaphoreType.DMA,
            pltpu.SemaphoreType.DMA,
        ],
    )
    return kern(h, src4, dst4)


def kernel(x, edge_index, W_in, b_in, W_rel1, b_rel1, W_root1,
           W_rel2, b_rel2, W_root2):
    e = edge_index.shape[1]
    info = plsc.get_sparse_core_info()
    ns = info.num_subcores
    cedges = _CB * _B
    assert e % (ns * cedges) == 0
    nchunks = e // (ns * cedges)

    src4 = edge_index[0].reshape(ns, nchunks, _CB, _B)
    dst4 = edge_index[1].reshape(ns, nchunks, _CB, _B)

    h0 = _tc_linear(x, W_in, b_in)
    p1 = _sc_segment_sum(h0, src4, dst4)
    h1 = _tc_combine([p1], h0, W_rel1, b_rel1, W_root1)
    p2 = _sc_segment_sum(h1, src4, dst4)
    h2 = _tc_combine([p2], h1, W_rel2, b_rel2, W_root2)
    return h2
